# baseline (speedup 1.0000x reference)
import jax
import jax.numpy as jnp
from jax.experimental import pallas as pl
from jax.experimental.pallas import tpu as pltpu
from jax.experimental.pallas import tpu_sc as plsc

_K = 8192
_C = 256
_NTOK = 8192
_TOK_TILE = 256
_K_TILE = 1024
_GATHER_WINDOW = 128


def _argmin_body(cn_ref, z_ref, cb_ref, idx_ref):
    zc = z_ref[0]
    z = zc.T
    zn = jnp.sum(z ** 2, axis=1, keepdims=True)
    z2 = z * 2.0
    ii = jax.lax.broadcasted_iota(
        jnp.int32, (_TOK_TILE, _K_TILE), 1).astype(jnp.float32)

    mins = None
    args = None
    for j in range(_K // _K_TILE):
        c = cb_ref[pl.ds(j * _K_TILE, _K_TILE), :]
        t3 = jax.lax.dot_general(
            z2, c, (((1,), (1,)), ((), ())),
            preferred_element_type=jnp.float32)
        cn = cn_ref[:, pl.ds(j * _K_TILE, _K_TILE)]
        d = (zn + cn) - t3
        lm = jnp.min(d, axis=1, keepdims=True)
        la = jnp.min(jnp.where(d == lm, ii, jnp.float32(_K)),
                     axis=1, keepdims=True)
        la = la.astype(jnp.int32) + j * _K_TILE
        if mins is None:
            mins, args = lm, la
        else:
            upd = lm < mins
            mins = jnp.where(upd, lm, mins)
            args = jnp.where(upd, la, args)
    idx_ref[0, 0, :] = args[:, 0]


def _compute_indices(cn, z2d, codebook):
    idx3 = pl.pallas_call(
        _argmin_body,
        grid=(_NTOK // _TOK_TILE,),
        in_specs=[
            pl.BlockSpec((1, _K), lambda i: (0, 0)),
            pl.BlockSpec((1, _C, _TOK_TILE), lambda i: (i // 4, 0, i % 4)),
            pl.BlockSpec((_K, _C), lambda i: (0, 0)),
        ],
        out_specs=pl.BlockSpec((1, 1, _TOK_TILE), lambda i: (i, 0, 0)),
        out_shape=jax.ShapeDtypeStruct(
            (_NTOK // _TOK_TILE, 1, _TOK_TILE), jnp.int32),
    )(cn, z2d, codebook)
    return idx3.reshape(_NTOK)


def _sc_gather(codebook, indices):
    idx2 = indices.reshape(1, _NTOK)
    mesh = plsc.VectorSubcoreMesh(
        core_axis_name="core", subcore_axis_name="subcore")

    @pl.kernel(out_type=jax.ShapeDtypeStruct((_NTOK, _C), jnp.float32),
               mesh=mesh)
    def gk(cb_hbm, i_hbm, o_hbm):
        def body(i_vmem, o_vmem):
            pltpu.sync_copy(cb_hbm.at[i_vmem.at[0]], o_vmem)

        pltpu.emit_pipeline(
            body,
            grid=(_NTOK // _GATHER_WINDOW,),
            in_specs=[pl.BlockSpec((1, _GATHER_WINDOW),
                                   index_map=lambda i: (0, i))],
            out_specs=[pl.BlockSpec((_GATHER_WINDOW, _C),
                                    index_map=lambda i: (i, 0))],
            core_axis_name=("core", "subcore"),
            dimension_semantics=(pltpu.PARALLEL,),
        )(i_hbm, o_hbm)

    return gk(codebook, idx2)


def _st_loss_body(q_ref, z_ref, qst_ref, acc_ref):
    @pl.when(pl.program_id(0) == 0)
    def _():
        acc_ref[...] = jnp.zeros((1, 1), jnp.float32)
    q = q_ref[...]
    zb = z_ref[...]
    d = q - zb
    qst_ref[...] = zb + d
    acc_ref[...] += jnp.sum(d * d).reshape(1, 1)


def _st_and_loss(q, z_raw):
    nblk = 8
    qst, tot = pl.pallas_call(
        _st_loss_body,
        grid=(nblk,),
        in_specs=[
            pl.BlockSpec((_NTOK // nblk, _C), lambda i: (i, 0)),
            pl.BlockSpec((_NTOK // nblk, _C), lambda i: (i, 0)),
        ],
        out_specs=[
            pl.BlockSpec((_NTOK // nblk, _C), lambda i: (i, 0)),
            pl.BlockSpec((1, 1), lambda i: (0, 0)),
        ],
        out_shape=[
            jax.ShapeDtypeStruct((_NTOK, _C), jnp.float32),
            jax.ShapeDtypeStruct((1, 1), jnp.float32),
        ],
    )(q, z_raw)
    loss = tot[0, 0] / jnp.float32(_NTOK * _C)
    return qst, loss


def kernel(z, codebook):
    B, C, H, W = z.shape
    z2d = z.reshape(B, C, H * W)
    cn = jnp.sum(codebook ** 2, axis=1).reshape(1, _K)

    indices = _compute_indices(cn, z2d, codebook)
    q = _sc_gather(codebook, indices)

    z_raw = z.reshape(_NTOK, C)
    qst, loss = _st_and_loss(q, z_raw)

    quantized_st = qst.reshape(z.shape)
    return (quantized_st, indices.reshape(B, H, W), loss, loss)
